# baseline (speedup 1.0000x reference)
import functools

import jax
import jax.numpy as jnp
from jax import lax
from jax.experimental import pallas as pl
from jax.experimental.pallas import tpu as pltpu
from jax.experimental.pallas import tpu_sc as plsc

B = 512
D = 4096
R = 64
N = 64
G = 8
STEPS = N // G


def _f16_bits_to_f32(h32):
    f32bits = ((h32 & 0x8000) << 16) | ((h32 & 0x7FFF) << 13)
    return pltpu.bitcast(f32bits, jnp.float32) * jnp.float32(2.0 ** 112)


def _mm_body(x_ref, a_ref, y_ref):
    a = _f16_bits_to_f32(a_ref[...].astype(jnp.int32)).astype(jnp.bfloat16)
    w = jnp.concatenate([a[i * D:(i + 1) * D, :] for i in range(G)], axis=1)
    yblk = lax.dot_general(
        x_ref[...], w, (((1,), (0,)), ((), ())),
        preferred_element_type=jnp.float32)
    for i in range(G):
        s = yblk[:, i * R:(i + 1) * R]
        y_ref[:, pl.ds(i * 2 * R, R)] = s
        y_ref[:, pl.ds(i * 2 * R + R, R)] = s


def _dense_all_adapters(x2d, lora_bits):
    return pl.pallas_call(
        _mm_body,
        grid=(STEPS,),
        in_specs=[
            pl.BlockSpec((B, D), lambda g: (0, 0)),
            pl.BlockSpec((G * D, R), lambda g: (g, 0)),
        ],
        out_specs=pl.BlockSpec((B, G * 2 * R), lambda g: (0, g)),
        out_shape=jax.ShapeDtypeStruct((B, N * 2 * R), jnp.float32),
    )(x2d, lora_bits)


_NC = 2
_NS = 16
_NW = _NC * _NS
_BPW = B // _NW


@functools.cache
def _make_route_gather():
    @functools.partial(
        pl.kernel,
        out_type=jax.ShapeDtypeStruct((B, 2 * R), jnp.float32),
        mesh=plsc.VectorSubcoreMesh(core_axis_name="c", subcore_axis_name="s"),
        scratch_types=[
            pltpu.VMEM((_BPW,), jnp.int32),
            pltpu.VMEM((_BPW,), jnp.int32),
            pltpu.VMEM((_BPW, 2 * R), jnp.float32),
            pltpu.SemaphoreType.DMA,
        ],
    )
    def _route_gather(y_hbm, wids_hbm, out_hbm, wids_v, idx_v, rows_v, sem):
        wid = lax.axis_index("s") * _NC + lax.axis_index("c")
        base = wid * _BPW
        pltpu.sync_copy(wids_hbm.at[pl.ds(base, _BPW)], wids_v)
        lane = lax.iota(jnp.int32, _BPW)
        idx_v[...] = (base + lane) * N + wids_v[...]
        pltpu.async_copy(y_hbm.at[idx_v], rows_v, sem).wait()
        pltpu.sync_copy(rows_v, out_hbm.at[pl.ds(base, _BPW)])

    return _route_gather


def kernel(x, wids, lora_A):
    x2d = x.reshape(B, D).astype(jnp.bfloat16)
    lora_bits = lax.bitcast_convert_type(lora_A, jnp.int16).reshape(N * D, R)
    y = _dense_all_adapters(x2d, lora_bits)
    h = _make_route_gather()(y.reshape(B * N, 2 * R), wids)
    return h[:, :R].astype(jnp.float16).reshape(B, 1, R)


# speedup vs baseline: 1.0920x; 1.0920x over previous
import functools

import jax
import jax.numpy as jnp
from jax import lax
from jax.experimental import pallas as pl
from jax.experimental.pallas import tpu as pltpu
from jax.experimental.pallas import tpu_sc as plsc

B = 512
D = 4096
R = 64
N = 64
G = 8
STEPS = N // G


def _f16_bits_to_f32(h32):
    f32bits = ((h32 & 0x8000) << 16) | ((h32 & 0x7FFF) << 13)
    return pltpu.bitcast(f32bits, jnp.float32) * jnp.float32(2.0 ** 112)


def _mm_body(x_ref, a_ref, y_ref):
    a = _f16_bits_to_f32(a_ref[...].astype(jnp.int32)).astype(jnp.bfloat16)
    w = jnp.concatenate([a[i * D:(i + 1) * D, :] for i in range(G)], axis=1)
    y_ref[...] = lax.dot_general(
        x_ref[...], w, (((1,), (0,)), ((), ())),
        preferred_element_type=jnp.float32)


def _dense_all_adapters(x2d, lora_bits):
    return pl.pallas_call(
        _mm_body,
        grid=(STEPS,),
        in_specs=[
            pl.BlockSpec((B, D), lambda g: (0, 0)),
            pl.BlockSpec((G * D, R), lambda g: (g, 0)),
        ],
        out_specs=pl.BlockSpec((B, G * R), lambda g: (0, g)),
        out_shape=jax.ShapeDtypeStruct((B, N * R), jnp.float32),
    )(x2d, lora_bits)


_NC = 2
_NS = 16
_NW = _NC * _NS
_BPW = B // _NW


_RPT = N * R // 128


@functools.cache
def _make_route_gather():
    @functools.partial(
        pl.kernel,
        out_type=jax.ShapeDtypeStruct((B, 2 * R), jnp.float32),
        mesh=plsc.VectorSubcoreMesh(core_axis_name="c", subcore_axis_name="s"),
        scratch_types=[
            pltpu.VMEM((_BPW,), jnp.int32),
            pltpu.VMEM((_BPW,), jnp.int32),
            pltpu.VMEM((_BPW, 2 * R), jnp.float32),
            pltpu.SemaphoreType.DMA,
        ],
    )
    def _route_gather(y_hbm, wids_hbm, out_hbm, wids_v, idx_v, rows_v, sem):
        wid = lax.axis_index("s") * _NC + lax.axis_index("c")
        base = wid * _BPW
        pltpu.sync_copy(wids_hbm.at[pl.ds(base, _BPW)], wids_v)
        lane = lax.iota(jnp.int32, _BPW)
        idx_v[...] = (base + lane) * _RPT + (wids_v[...] >> 1)
        pltpu.async_copy(y_hbm.at[idx_v], rows_v, sem).wait()
        pltpu.sync_copy(rows_v, out_hbm.at[pl.ds(base, _BPW)])

    return _route_gather


def kernel(x, wids, lora_A):
    x2d = x.reshape(B, D).astype(jnp.bfloat16)
    lora_bits = lax.bitcast_convert_type(lora_A, jnp.int16).reshape(N * D, R)
    y = _dense_all_adapters(x2d, lora_bits)
    h = _make_route_gather()(
        y.reshape(B * N * R // 128, 128), wids)
    out = jnp.where((wids % 2 == 1)[:, None], h[:, R:], h[:, :R])
    return out.astype(jnp.float16).reshape(B, 1, R)
